# Initial kernel scaffold; baseline (speedup 1.0000x reference)
import jax
import jax.numpy as jnp
from jax.experimental import pallas as pl


def kernel(x, edge_index, W0, a0, W1, a1, W2, a2, W3, a3, W_out, a_out):
    raise NotImplementedError("write your pallas kernel here")


# speedup vs baseline: 65.7143x; 65.7143x over previous
import functools

import jax
import jax.numpy as jnp
from jax import lax
from jax.experimental import pallas as pl
from jax.experimental.pallas import tpu as pltpu
import jax.experimental.pallas.tpu_sc as plsc

N = 10000
F = 128
HID = 32
NH = 4
NCLS = 32
ALPHA = 0.2
E = 320000

UW = 144
VW = 48

NWK = 32
B = 128
T = 79
EPW = T * B
EPAD = NWK * EPW
SROWS = 10112

RB = 400
GRID = N // RB


def _make_sc_scatter(width):
    zrows = SROWS // 16
    olast = N - 15 * zrows
    mesh = plsc.VectorSubcoreMesh(core_axis_name="c", subcore_axis_name="s")
    out_sd = jax.ShapeDtypeStruct((N, width), jnp.float32)

    @functools.partial(
        pl.kernel,
        out_type=(out_sd, out_sd),
        mesh=mesh,
        scratch_types=[
            pltpu.VMEM((T, B), jnp.int32),
            pltpu.VMEM((T, B), jnp.int32),
            pltpu.VMEM((B, width), jnp.float32),
            pltpu.VMEM_SHARED((SROWS, width), jnp.float32),
            pltpu.SemaphoreType.DMA,
        ],
        compiler_params=pltpu.CompilerParams(use_tc_tiling_on_sc=False),
    )
    def sc_scatter(table, src_hbm, dst_hbm, out0, out1, src_v, dst_v, buf, acc, sem):
        c = lax.axis_index("c")
        s = lax.axis_index("s")
        wid = c * 16 + s

        def zrow(r, carry):
            def zcol(j, carry2):
                buf[r, pl.ds(j * 16, 16)] = jnp.zeros((16,), jnp.float32)
                return carry2
            return lax.fori_loop(0, width // 16, zcol, carry)
        lax.fori_loop(0, B, zrow, 0)

        zbase = s * zrows
        for k in range(zrows // B):
            pltpu.sync_copy(buf, acc.at[pl.ds(zbase + k * B, B)])
        rem = zrows % B
        if rem:
            pltpu.sync_copy(buf.at[pl.ds(0, rem)],
                            acc.at[pl.ds(zbase + (zrows // B) * B, rem)])
        plsc.subcore_barrier()

        pltpu.sync_copy(src_hbm.at[wid], src_v)
        pltpu.sync_copy(dst_hbm.at[wid], dst_v)

        def body(t, carry):
            pltpu.async_copy(table.at[dst_v.at[t]], buf, sem).wait()
            pltpu.sync_copy(buf, acc.at[src_v.at[t]], add=True)
            return carry
        lax.fori_loop(0, T, body, 0)

        plsc.subcore_barrier()
        ob = s * zrows

        @pl.when((c == 0) & (s < 15))
        def _():
            pltpu.sync_copy(acc.at[pl.ds(ob, zrows)], out0.at[pl.ds(ob, zrows)])

        @pl.when((c == 0) & (s == 15))
        def _():
            pltpu.sync_copy(acc.at[pl.ds(15 * zrows, olast)],
                            out0.at[pl.ds(15 * zrows, olast)])

        @pl.when((c == 1) & (s < 15))
        def _():
            pltpu.sync_copy(acc.at[pl.ds(ob, zrows)], out1.at[pl.ds(ob, zrows)])

        @pl.when((c == 1) & (s == 15))
        def _():
            pltpu.sync_copy(acc.at[pl.ds(15 * zrows, olast)],
                            out1.at[pl.ds(15 * zrows, olast)])

    return sc_scatter


_sc_scatter_u = _make_sc_scatter(UW)
_sc_scatter_v = _make_sc_scatter(VW)


def _tc_dense1(x, wcat, a2):
    def body(x_ref, w_ref, a_ref, wh_ref, e2_ref):
        z = jnp.dot(x_ref[...], w_ref[...], preferred_element_type=jnp.float32)
        wh = jnp.where(z > 0, z, ALPHA * z)
        wh_ref[...] = wh
        e2_ref[...] = jnp.dot(wh, a_ref[...], preferred_element_type=jnp.float32)

    return pl.pallas_call(
        body,
        grid=(GRID,),
        in_specs=[
            pl.BlockSpec((RB, F), lambda i: (i, 0)),
            pl.BlockSpec((F, F), lambda i: (0, 0)),
            pl.BlockSpec((F, 8), lambda i: (0, 0)),
        ],
        out_specs=[
            pl.BlockSpec((RB, F), lambda i: (i, 0)),
            pl.BlockSpec((RB, 8), lambda i: (i, 0)),
        ],
        out_shape=[
            jax.ShapeDtypeStruct((N, F), jnp.float32),
            jax.ShapeDtypeStruct((N, 8), jnp.float32),
        ],
    )(x, wcat, a2)


def _tc_build_u(wh, e2):
    def body(wh_ref, e2b_ref, e2f_ref, u_ref):
        cmax = jnp.max(e2f_ref[...], axis=0)
        g = jnp.exp(e2b_ref[...] - cmax[None, :])
        scale = jnp.concatenate(
            [jnp.broadcast_to(g[:, h:h + 1], (RB, HID)) for h in range(NH)], axis=1)
        u_ref[...] = jnp.concatenate(
            [wh_ref[...] * scale, g[:, :NH],
             jnp.zeros((RB, UW - F - NH), jnp.float32)], axis=1)

    return pl.pallas_call(
        body,
        grid=(GRID,),
        in_specs=[
            pl.BlockSpec((RB, F), lambda i: (i, 0)),
            pl.BlockSpec((RB, 8), lambda i: (i, 0)),
            pl.BlockSpec((N, 8), lambda i: (0, 0)),
        ],
        out_specs=pl.BlockSpec((RB, UW), lambda i: (i, 0)),
        out_shape=jax.ShapeDtypeStruct((N, UW), jnp.float32),
    )(wh, e2, e2)


def _tc_dense2(pa0, pa1, wout, a2b):
    def body(p0_ref, p1_ref, w_ref, a_ref, wh2_ref, e2b_ref):
        sacc = p0_ref[...] + p1_ref[...]
        num = sacc[:, :F]
        dinv = 1.0 / jnp.maximum(sacc[:, F:F + NH], 1e-16)
        scale = jnp.concatenate(
            [jnp.broadcast_to(dinv[:, h:h + 1], (RB, HID)) for h in range(NH)], axis=1)
        hp = num * scale
        hcat = jnp.where(hp > 0, hp, jnp.exp(hp) - 1.0)
        z = jnp.dot(hcat, w_ref[...], preferred_element_type=jnp.float32)
        wh2 = jnp.where(z > 0, z, ALPHA * z)
        wh2_ref[...] = wh2
        e2b_ref[...] = jnp.dot(wh2, a_ref[...], preferred_element_type=jnp.float32)

    return pl.pallas_call(
        body,
        grid=(GRID,),
        in_specs=[
            pl.BlockSpec((RB, UW), lambda i: (i, 0)),
            pl.BlockSpec((RB, UW), lambda i: (i, 0)),
            pl.BlockSpec((F, NCLS), lambda i: (0, 0)),
            pl.BlockSpec((NCLS, 8), lambda i: (0, 0)),
        ],
        out_specs=[
            pl.BlockSpec((RB, NCLS), lambda i: (i, 0)),
            pl.BlockSpec((RB, 8), lambda i: (i, 0)),
        ],
        out_shape=[
            jax.ShapeDtypeStruct((N, NCLS), jnp.float32),
            jax.ShapeDtypeStruct((N, 8), jnp.float32),
        ],
    )(pa0, pa1, wout, a2b)


def _tc_build_v(wh2, e2b):
    def body(wh2_ref, e2b_ref, e2f_ref, v_ref):
        cmax = jnp.max(e2f_ref[...][:, 0])
        g = jnp.exp(e2b_ref[...][:, 0:1] - cmax)
        v_ref[...] = jnp.concatenate(
            [wh2_ref[...] * g, g,
             jnp.zeros((RB, VW - NCLS - 1), jnp.float32)], axis=1)

    return pl.pallas_call(
        body,
        grid=(GRID,),
        in_specs=[
            pl.BlockSpec((RB, NCLS), lambda i: (i, 0)),
            pl.BlockSpec((RB, 8), lambda i: (i, 0)),
            pl.BlockSpec((N, 8), lambda i: (0, 0)),
        ],
        out_specs=pl.BlockSpec((RB, VW), lambda i: (i, 0)),
        out_shape=jax.ShapeDtypeStruct((N, VW), jnp.float32),
    )(wh2, e2b, e2b)


def _tc_final(pb0, pb1):
    def body(p0_ref, p1_ref, o_ref):
        sacc = p0_ref[...] + p1_ref[...]
        den = jnp.maximum(sacc[:, NCLS:NCLS + 1], 1e-16)
        o = sacc[:, :NCLS] / den
        o = jnp.where(o > 0, o, jnp.exp(o) - 1.0)
        m = jnp.max(o, axis=1, keepdims=True)
        ex = jnp.exp(o - m)
        o_ref[...] = o - (jnp.log(jnp.sum(ex, axis=1, keepdims=True)) + m)

    return pl.pallas_call(
        body,
        grid=(GRID,),
        in_specs=[
            pl.BlockSpec((RB, VW), lambda i: (i, 0)),
            pl.BlockSpec((RB, VW), lambda i: (i, 0)),
        ],
        out_specs=pl.BlockSpec((RB, NCLS), lambda i: (i, 0)),
        out_shape=jax.ShapeDtypeStruct((N, NCLS), jnp.float32),
    )(pb0, pb1)


def kernel(x, edge_index, W0, a0, W1, a1, W2, a2, W3, a3, W_out, a_out):
    x = x.astype(jnp.float32)
    ei = edge_index.astype(jnp.int32)
    pad = EPAD - E
    srcp = jnp.concatenate([ei[0], jnp.full((pad,), N, jnp.int32)]).reshape(NWK, T, B)
    dstp = jnp.concatenate([ei[1], jnp.zeros((pad,), jnp.int32)]).reshape(NWK, T, B)

    wcat = jnp.concatenate([W0, W1, W2, W3], axis=1)
    a2cols = [
        jnp.zeros((F, 1), jnp.float32).at[h * HID:(h + 1) * HID].set(a[HID:2 * HID])
        for h, a in enumerate((a0, a1, a2, a3))
    ]
    a2 = jnp.concatenate(a2cols + [jnp.zeros((F, 4), jnp.float32)], axis=1)
    a2b = jnp.concatenate([a_out[NCLS:2 * NCLS], jnp.zeros((NCLS, 7), jnp.float32)],
                          axis=1)

    wh, e2 = _tc_dense1(x, wcat, a2)
    u = _tc_build_u(wh, e2)
    pa0, pa1 = _sc_scatter_u(u, srcp, dstp)
    wh2, e2b = _tc_dense2(pa0, pa1, W_out, a2b)
    v = _tc_build_v(wh2, e2b)
    pb0, pb1 = _sc_scatter_v(v, srcp, dstp)
    return _tc_final(pb0, pb1)
